# baseline (speedup 1.0000x reference)
import jax
import jax.numpy as jnp
from jax import lax
from jax.experimental import pallas as pl
from jax.experimental.pallas import tpu as pltpu
from jax.experimental.pallas import tpu_sc as plsc

VOCAB = 30522
DIM = 768
EPS = 1e-5
LANES = 16
DCH = DIM // LANES

NC = 2
NS = 16
NW = NC * NS

SENTS = 4096
SLEN = 50
B_TOTAL = SENTS * SLEN
RPW = B_TOTAL // NW
CHUNK = 32
NCHUNK = RPW // CHUNK
NBUF = 4
RGRP = 16


def _lane_sum(v):
    for k in (8, 4, 2, 1):
        idx = (lax.iota(jnp.int32, LANES) + k) & (LANES - 1)
        v = v + v.at[idx].get(mode="promise_in_bounds", unique_indices=True)
    return v[0]


def _rsqrt(x):
    i = lax.bitcast_convert_type(x, jnp.int32)
    i = jnp.int32(0x5F3759DF) - (i >> 1)
    y = lax.bitcast_convert_type(i, jnp.float32)
    for _ in range(3):
        y = y * (1.5 - 0.5 * x * y * y)
    return y


def _row_stats(buf, r):
    acc = [jnp.zeros((LANES,), jnp.float32) for _ in range(4)]
    acc2 = [jnp.zeros((LANES,), jnp.float32) for _ in range(4)]
    for d in range(DCH):
        v = buf[r, pl.ds(LANES * d, LANES)]
        acc[d % 4] = acc[d % 4] + v
        acc2[d % 4] = acc2[d % 4] + v * v
    s = (acc[0] + acc[1]) + (acc[2] + acc[3])
    ss = (acc2[0] + acc2[1]) + (acc2[2] + acc2[3])
    mean = _lane_sum(s) * (1.0 / DIM)
    var = _lane_sum(ss) * (1.0 / DIM) - mean * mean
    return mean, _rsqrt(var + EPS)


def _ln_chunk(buf, boff, wv, bv):
    def grp_body(g, _):
        r0 = boff + g * RGRP
        stats = [_row_stats(buf, r0 + k) for k in range(RGRP)]
        for d in range(DCH):
            sl = pl.ds(LANES * d, LANES)
            wd = wv[sl]
            bd = bv[sl]
            for k in range(RGRP):
                m, a = stats[k]
                v = buf[r0 + k, sl]
                buf[r0 + k, sl] = ((v - m) * a) * wd + bd
        return 0

    lax.fori_loop(0, CHUNK // RGRP, grp_body, 0)


def _sc_body(ids_hbm, table_hbm, w_hbm, b_hbm, out_hbm,
             idx_v, buf_all, wv, bv, gsems, osems):
    wid = lax.axis_index("s") * NC + lax.axis_index("c")

    pltpu.sync_copy(ids_hbm.at[wid], idx_v)
    pltpu.sync_copy(w_hbm, wv)
    pltpu.sync_copy(b_hbm, bv)

    def bview(b):
        return buf_all.at[pl.ds(b * CHUNK, CHUNK)]

    def start_gather(c, b):
        pltpu.async_copy(table_hbm.at[idx_v.at[c]], bview(b), gsems.at[b])

    def wait_gather(c, b):
        pltpu.make_async_copy(
            table_hbm.at[idx_v.at[c]], bview(b), gsems.at[b]).wait()

    def start_out(c, b):
        dst = out_hbm.at[pl.ds(wid * RPW + c * CHUNK, CHUNK)]
        pltpu.async_copy(bview(b), dst, osems.at[b])

    def wait_out(b):
        dst = out_hbm.at[pl.ds(0, CHUNK)]
        pltpu.make_async_copy(bview(b), dst, osems.at[b]).wait()

    for c in range(NBUF - 1):
        start_gather(c, c)

    def step(c, _):
        b = c & (NBUF - 1)
        wait_gather(c, b)
        _ln_chunk(buf_all, b * CHUNK, wv, bv)
        start_out(c, b)
        cp = c + NBUF - 1

        @pl.when(cp < NCHUNK)
        def _():
            bp = cp & (NBUF - 1)

            @pl.when(cp >= NBUF)
            def _():
                wait_out(bp)

            start_gather(cp, bp)

        return 0

    lax.fori_loop(0, NCHUNK, step, 0)

    for b in range(NBUF):
        wait_out(b)


@jax.jit
def _run(ids, table, ln_weight, ln_bias):
    mesh = plsc.VectorSubcoreMesh(core_axis_name="c", subcore_axis_name="s")
    f = pl.kernel(
        _sc_body,
        out_type=jax.ShapeDtypeStruct((B_TOTAL, DIM), jnp.float32),
        mesh=mesh,
        scratch_types=[
            pltpu.VMEM((NCHUNK, CHUNK), jnp.int32),
            pltpu.VMEM((NBUF * CHUNK, DIM), jnp.float32),
            pltpu.VMEM((DIM,), jnp.float32),
            pltpu.VMEM((DIM,), jnp.float32),
            pltpu.SemaphoreType.DMA((NBUF,)),
            pltpu.SemaphoreType.DMA((NBUF,)),
        ],
    )
    return f(ids, table, ln_weight, ln_bias)


def kernel(input_ids, table, ln_weight, ln_bias):
    ids = input_ids.T.reshape(NW, NCHUNK, CHUNK)
    out = _run(ids, table, ln_weight, ln_bias)
    return jnp.transpose(out.reshape(SLEN, SENTS, DIM), (1, 0, 2))


# speedup vs baseline: 2.0250x; 2.0250x over previous
import jax
import jax.numpy as jnp
from jax import lax
from jax.experimental import pallas as pl
from jax.experimental.pallas import tpu as pltpu
from jax.experimental.pallas import tpu_sc as plsc

VOCAB = 30522
DIM = 768
EPS = 1e-5
LANES = 16
DCH = DIM // LANES

NC = 2
NS = 16
NW = NC * NS

SENTS = 4096
SLEN = 50
B_TOTAL = SENTS * SLEN
RPW = B_TOTAL // NW
CHUNK = 32
NCHUNK = RPW // CHUNK
NBUF = 4
RGRP = 8


def _lane_sum(v):
    for k in (8, 4, 2, 1):
        idx = (lax.iota(jnp.int32, LANES) + k) & (LANES - 1)
        v = v + v.at[idx].get(mode="promise_in_bounds", unique_indices=True)
    return v[0]


def _rsqrt(x):
    i = lax.bitcast_convert_type(x, jnp.int32)
    i = jnp.int32(0x5F3759DF) - (i >> 1)
    y = lax.bitcast_convert_type(i, jnp.float32)
    for _ in range(3):
        y = y * (1.5 - 0.5 * x * y * y)
    return y


def _row_stats(buf, r):
    acc = [jnp.zeros((LANES,), jnp.float32) for _ in range(4)]
    acc2 = [jnp.zeros((LANES,), jnp.float32) for _ in range(4)]
    for d in range(DCH):
        v = buf[r, pl.ds(LANES * d, LANES)]
        acc[d % 4] = acc[d % 4] + v
        acc2[d % 4] = acc2[d % 4] + v * v
    s = (acc[0] + acc[1]) + (acc[2] + acc[3])
    ss = (acc2[0] + acc2[1]) + (acc2[2] + acc2[3])
    mean = _lane_sum(s) * (1.0 / DIM)
    var = _lane_sum(ss) * (1.0 / DIM) - mean * mean
    return mean, _rsqrt(var + EPS)


def _ln_chunk(buf, boff, wv, bv):
    def grp_body(g, _):
        r0 = boff + g * RGRP
        stats = [_row_stats(buf, r0 + k) for k in range(RGRP)]
        for d in range(DCH):
            sl = pl.ds(LANES * d, LANES)
            wd = wv[sl]
            bd = bv[sl]
            for k in range(RGRP):
                m, a = stats[k]
                v = buf[r0 + k, sl]
                buf[r0 + k, sl] = ((v - m) * a) * wd + bd
        return 0

    lax.fori_loop(0, CHUNK // RGRP, grp_body, 0)


def _sc_body(ids_hbm, table_hbm, w_hbm, b_hbm, out_hbm,
             idx_v, buf_all, wv, bv, gsems, osems):
    wid = lax.axis_index("s") * NC + lax.axis_index("c")

    pltpu.sync_copy(ids_hbm.at[wid], idx_v)
    pltpu.sync_copy(w_hbm, wv)
    pltpu.sync_copy(b_hbm, bv)

    def bview(b):
        return buf_all.at[pl.ds(b * CHUNK, CHUNK)]

    def start_gather(c, b):
        pltpu.async_copy(table_hbm.at[idx_v.at[c]], bview(b), gsems.at[b])

    def wait_gather(c, b):
        pltpu.make_async_copy(
            table_hbm.at[idx_v.at[c]], bview(b), gsems.at[b]).wait()

    def start_out(c, b):
        dst = out_hbm.at[pl.ds(wid * RPW + c * CHUNK, CHUNK)]
        pltpu.async_copy(bview(b), dst, osems.at[b])

    def wait_out(b):
        dst = out_hbm.at[pl.ds(0, CHUNK)]
        pltpu.make_async_copy(bview(b), dst, osems.at[b]).wait()

    for c in range(NBUF - 1):
        start_gather(c, c)

    def step(c, _):
        b = c & (NBUF - 1)
        wait_gather(c, b)
        _ln_chunk(buf_all, b * CHUNK, wv, bv)
        start_out(c, b)
        cp = c + NBUF - 1

        @pl.when(cp < NCHUNK)
        def _():
            bp = cp & (NBUF - 1)

            @pl.when(cp >= NBUF)
            def _():
                wait_out(bp)

            start_gather(cp, bp)

        return 0

    lax.fori_loop(0, NCHUNK, step, 0)

    for b in range(NBUF):
        wait_out(b)


@jax.jit
def _run(ids, table, ln_weight, ln_bias):
    mesh = plsc.VectorSubcoreMesh(core_axis_name="c", subcore_axis_name="s")
    f = pl.kernel(
        _sc_body,
        out_type=jax.ShapeDtypeStruct((B_TOTAL, DIM), jnp.float32),
        mesh=mesh,
        scratch_types=[
            pltpu.VMEM((NCHUNK, CHUNK), jnp.int32),
            pltpu.VMEM((NBUF * CHUNK, DIM), jnp.float32),
            pltpu.VMEM((DIM,), jnp.float32),
            pltpu.VMEM((DIM,), jnp.float32),
            pltpu.SemaphoreType.DMA((NBUF,)),
            pltpu.SemaphoreType.DMA((NBUF,)),
        ],
    )
    return f(ids, table, ln_weight, ln_bias)


def kernel(input_ids, table, ln_weight, ln_bias):
    ids = input_ids.T.reshape(NW, NCHUNK, CHUNK)
    out = _run(ids, table, ln_weight, ln_bias)
    return jnp.transpose(out.reshape(SLEN, SENTS, DIM), (1, 0, 2))
